# baseline (speedup 1.0000x reference)
import jax
import jax.numpy as jnp
from jax.experimental import pallas as pl
from jax.experimental.pallas import tpu as pltpu

N, D, H = 4096, 128, 128
H2 = 32
BLK = 512
SUB = 128
T = N // BLK


def _gnn_kernel(x_ref, adj_ref, Win_ref, bin_ref, W1_ref, b1_ref, W2_ref,
                b2_ref, Wout_ref, bout_ref, out_ref, cnt_ref,
                gext_ref, s1_ref, yext_ref, adj16_ref, nspk_ref):
    p = pl.program_id(0)
    t = pl.program_id(1)

    @pl.when(jnp.logical_and(p == 0, t == 0))
    def _init():
        h = jnp.tanh(
            jax.lax.dot_general(x_ref[...].astype(jnp.bfloat16),
                                Win_ref[...].astype(jnp.bfloat16),
                                (((1,), (1,)), ((), ())),
                                preferred_element_type=jnp.float32)
            + bin_ref[...])
        g = jax.lax.dot_general(h.astype(jnp.bfloat16),
                                W1_ref[...].astype(jnp.bfloat16),
                                (((1,), (1,)), ((), ())),
                                preferred_element_type=jnp.float32)
        gext_ref[...] = g.astype(jnp.bfloat16)
        cnt_ref[...] = jnp.zeros((1, 1), jnp.float32)
        nspk_ref[0] = 0.0

    @pl.when(p == 0)
    def _phase0():
        rows = pl.ds(t * BLK, BLK)
        a16 = adj_ref[...].astype(jnp.bfloat16)
        adj16_ref[rows, :] = a16
        deg = jnp.sum(adj_ref[...], axis=1, keepdims=True)
        agg_g = jnp.dot(a16, gext_ref[...], preferred_element_type=jnp.float32)
        z1 = jnp.where(deg > 0, agg_g / jnp.maximum(deg, 1.0),
                       gext_ref[rows, :].astype(jnp.float32)) + b1_ref[...]
        s1 = z1 * 0.5 >= 1.0
        s1_ref[rows, :] = s1.astype(jnp.bfloat16)
        ns = jnp.sum(s1.astype(jnp.float32))
        cnt_ref[...] += ns.reshape(1, 1)
        nspk_ref[0] += ns

    spiking = nspk_ref[0] > 0.0

    @pl.when(jnp.logical_and(jnp.logical_and(p == 1, t == 0), spiking))
    def _mid():
        y = jax.lax.dot_general(s1_ref[...], W2_ref[...].astype(jnp.bfloat16),
                                (((1,), (1,)), ((), ())),
                                preferred_element_type=jnp.float32)
        yext_ref[:, :H2] = y.astype(jnp.bfloat16)
        ones_col = (jax.lax.broadcasted_iota(jnp.int32, (N, H - H2), 1) == 0)
        yext_ref[:, H2:] = ones_col.astype(jnp.bfloat16)

    @pl.when(jnp.logical_and(p == 1, spiking))
    def _phase1():
        rows = pl.ds(t * BLK, BLK)
        a16 = adj16_ref[rows, :]
        r = jnp.dot(a16, yext_ref[...], preferred_element_type=jnp.float32)
        agg_y = r[:, :H2]
        deg = r[:, H2:H2 + 1]
        z2 = jnp.where(deg > 0, agg_y / jnp.maximum(deg, 1.0),
                       yext_ref[rows, :H2].astype(jnp.float32)) + b2_ref[...]
        s2 = (z2 * 0.5 >= 1.0).astype(jnp.float32)
        o = jax.lax.dot_general(s2, Wout_ref[...], (((1,), (1,)), ((), ())),
                                preferred_element_type=jnp.float32) + bout_ref[...]
        out_ref[...] = o
        cnt_ref[...] += jnp.sum(s2).reshape(1, 1)

    @pl.when(jnp.logical_and(p == 1, jnp.logical_not(spiking)))
    def _phase1_nospike():
        s2 = (b2_ref[...] * 0.5 >= 1.0).astype(jnp.float32)
        o = jax.lax.dot_general(s2, Wout_ref[...], (((1,), (1,)), ((), ())),
                                preferred_element_type=jnp.float32) + bout_ref[...]
        out_ref[...] = jnp.broadcast_to(o, (BLK, 4))
        cnt_ref[...] += (jnp.sum(s2) * BLK).reshape(1, 1)


@jax.jit
def _forward(x, adj_matrix, W_in, b_in, W1, b1, W2, b2, W_out, b_out):
    return pl.pallas_call(
        _gnn_kernel,
        grid=(2, T),
        in_specs=[
            pl.BlockSpec((N, D), lambda p, t: (0, 0)),
            pl.BlockSpec((BLK, N), lambda p, t: (jnp.where(p == 0, t, T - 1), 0)),
            pl.BlockSpec((H, D), lambda p, t: (0, 0)),
            pl.BlockSpec((1, H), lambda p, t: (0, 0)),
            pl.BlockSpec((H, H), lambda p, t: (0, 0)),
            pl.BlockSpec((1, H), lambda p, t: (0, 0)),
            pl.BlockSpec((H2, H), lambda p, t: (0, 0)),
            pl.BlockSpec((1, H2), lambda p, t: (0, 0)),
            pl.BlockSpec((4, H2), lambda p, t: (0, 0)),
            pl.BlockSpec((1, 4), lambda p, t: (0, 0)),
        ],
        out_specs=[
            pl.BlockSpec((BLK, 4), lambda p, t: (p * t, 0)),
            pl.BlockSpec((1, 1), lambda p, t: (0, 0)),
        ],
        out_shape=[
            jax.ShapeDtypeStruct((N, 4), jnp.float32),
            jax.ShapeDtypeStruct((1, 1), jnp.float32),
        ],
        scratch_shapes=[
            pltpu.VMEM((N, H), jnp.bfloat16),
            pltpu.VMEM((N, H), jnp.bfloat16),
            pltpu.VMEM((N, H), jnp.bfloat16),
            pltpu.VMEM((N, N), jnp.bfloat16),
            pltpu.SMEM((1,), jnp.float32),
        ],
    )(x, adj_matrix, W_in, b_in, W1, b1, W2, b2, W_out, b_out)


def kernel(x, adj_matrix, W_in, b_in, W1, b1, W2, b2, W_out, b_out):
    out, cnt = _forward(x, adj_matrix, W_in, b_in.reshape(1, -1), W1,
                        b1.reshape(1, -1), W2, b2.reshape(1, -1), W_out,
                        b_out.reshape(1, -1))
    total_spikes = cnt[0, 0]
    energy_pj = total_spikes * 1.0
    sparsity = 1.0 - total_spikes / (x.shape[0] * 128)
    return out, total_spikes, energy_pj, sparsity


# speedup vs baseline: 1.0079x; 1.0079x over previous
import jax
import jax.numpy as jnp
from jax.experimental import pallas as pl
from jax.experimental.pallas import tpu as pltpu

N, D, H = 4096, 128, 128
H2 = 32
BLK = 512
T = N // BLK


def _gnn_kernel(x_ref, adj_ref, Win_ref, bin_ref, W1_ref, b1_ref, W2_ref,
                b2_ref, Wout_ref, bout_ref, out_ref, cnt_ref,
                gext_ref, s1_ref, yext_ref, adj16_ref, nspk_ref):
    p = pl.program_id(0)
    t = pl.program_id(1)

    @pl.when(jnp.logical_and(p == 0, t == 0))
    def _init():
        h = jnp.tanh(
            jax.lax.dot_general(x_ref[...].astype(jnp.bfloat16),
                                Win_ref[...].astype(jnp.bfloat16),
                                (((1,), (1,)), ((), ())),
                                preferred_element_type=jnp.float32)
            + bin_ref[...])
        g = jax.lax.dot_general(h.astype(jnp.bfloat16),
                                W1_ref[...].astype(jnp.bfloat16),
                                (((1,), (1,)), ((), ())),
                                preferred_element_type=jnp.float32)
        gext_ref[...] = g.astype(jnp.bfloat16)
        cnt_ref[...] = jnp.zeros((1, 1), jnp.float32)
        nspk_ref[0] = 0.0

    @pl.when(p == 0)
    def _phase0():
        rows = pl.ds(t * BLK, BLK)
        a16 = adj_ref[...].astype(jnp.bfloat16)
        adj16_ref[rows, :] = a16
        deg = jnp.sum(adj_ref[...], axis=1, keepdims=True)
        agg_g = jnp.dot(a16, gext_ref[...], preferred_element_type=jnp.float32)
        z1 = jnp.where(deg > 0, agg_g / jnp.maximum(deg, 1.0),
                       gext_ref[rows, :].astype(jnp.float32)) + b1_ref[...]
        s1 = z1 * 0.5 >= 1.0
        s1_ref[rows, :] = s1.astype(jnp.bfloat16)
        ns = jnp.sum(s1.astype(jnp.float32))
        cnt_ref[...] += ns.reshape(1, 1)
        nspk_ref[0] += ns

    spiking = nspk_ref[0] > 0.0

    @pl.when(jnp.logical_and(jnp.logical_and(p == 1, t == 0), spiking))
    def _mid():
        y = jax.lax.dot_general(s1_ref[...], W2_ref[...].astype(jnp.bfloat16),
                                (((1,), (1,)), ((), ())),
                                preferred_element_type=jnp.float32)
        yext_ref[:, :H2] = y.astype(jnp.bfloat16)
        ones_col = (jax.lax.broadcasted_iota(jnp.int32, (N, H - H2), 1) == 0)
        yext_ref[:, H2:] = ones_col.astype(jnp.bfloat16)

    @pl.when(jnp.logical_and(p == 1, spiking))
    def _phase1():
        rows = pl.ds(t * BLK, BLK)
        a16 = adj16_ref[rows, :]
        r = jnp.dot(a16, yext_ref[...], preferred_element_type=jnp.float32)
        agg_y = r[:, :H2]
        deg = r[:, H2:H2 + 1]
        z2 = jnp.where(deg > 0, agg_y / jnp.maximum(deg, 1.0),
                       yext_ref[rows, :H2].astype(jnp.float32)) + b2_ref[...]
        s2 = (z2 * 0.5 >= 1.0).astype(jnp.float32)
        o = jax.lax.dot_general(s2, Wout_ref[...], (((1,), (1,)), ((), ())),
                                preferred_element_type=jnp.float32) + bout_ref[...]
        out_ref[...] = o
        cnt_ref[...] += jnp.sum(s2).reshape(1, 1)

    @pl.when(jnp.logical_and(p == 1, jnp.logical_not(spiking)))
    def _phase1_nospike():
        s2 = (b2_ref[...] * 0.5 >= 1.0).astype(jnp.float32)
        o = jax.lax.dot_general(s2, Wout_ref[...], (((1,), (1,)), ((), ())),
                                preferred_element_type=jnp.float32) + bout_ref[...]
        out_ref[...] = jnp.broadcast_to(o, (BLK, 4))
        cnt_ref[...] += (jnp.sum(s2) * BLK).reshape(1, 1)


@jax.jit
def _forward(x, adj_matrix, W_in, b_in, W1, b1, W2, b2, W_out, b_out):
    return pl.pallas_call(
        _gnn_kernel,
        grid=(2, T),
        in_specs=[
            pl.BlockSpec((N, D), lambda p, t: (0, 0)),
            pl.BlockSpec((BLK, N), lambda p, t: (jnp.where(p == 0, t, T - 1), 0)),
            pl.BlockSpec((H, D), lambda p, t: (0, 0)),
            pl.BlockSpec((1, H), lambda p, t: (0, 0)),
            pl.BlockSpec((H, H), lambda p, t: (0, 0)),
            pl.BlockSpec((1, H), lambda p, t: (0, 0)),
            pl.BlockSpec((H2, H), lambda p, t: (0, 0)),
            pl.BlockSpec((1, H2), lambda p, t: (0, 0)),
            pl.BlockSpec((4, H2), lambda p, t: (0, 0)),
            pl.BlockSpec((1, 4), lambda p, t: (0, 0)),
        ],
        out_specs=[
            pl.BlockSpec((BLK, 4), lambda p, t: (p * t, 0)),
            pl.BlockSpec((1, 1), lambda p, t: (0, 0)),
        ],
        out_shape=[
            jax.ShapeDtypeStruct((N, 4), jnp.float32),
            jax.ShapeDtypeStruct((1, 1), jnp.float32),
        ],
        scratch_shapes=[
            pltpu.VMEM((N, H), jnp.bfloat16),
            pltpu.VMEM((N, H), jnp.bfloat16),
            pltpu.VMEM((N, H), jnp.bfloat16),
            pltpu.VMEM((N, N), jnp.bfloat16),
            pltpu.SMEM((1,), jnp.float32),
        ],
    )(x, adj_matrix, W_in, b_in, W1, b1, W2, b2, W_out, b_out)


def kernel(x, adj_matrix, W_in, b_in, W1, b1, W2, b2, W_out, b_out):
    out, cnt = _forward(x, adj_matrix, W_in, b_in.reshape(1, -1), W1,
                        b1.reshape(1, -1), W2, b2.reshape(1, -1), W_out,
                        b_out.reshape(1, -1))
    total_spikes = cnt[0, 0]
    energy_pj = total_spikes * 1.0
    sparsity = 1.0 - total_spikes / (x.shape[0] * 128)
    return out, total_spikes, energy_pj, sparsity
